# baseline (speedup 1.0000x reference)
import functools

import jax
import jax.numpy as jnp
from jax import lax
from jax.experimental import pallas as pl
from jax.experimental.pallas import tpu as pltpu
from jax.experimental.pallas import tpu_sc as plsc

B, S, H, NH, DH = 16, 512, 256, 4, 64
NCAT, NPROD = 100, 1000
BLK = 128
NBLK = 8
P = NBLK * BLK
NEG = -1e9
SCALE = 1.0 / (DH ** 0.5)


def _ln(x, g, b):
    m = x.mean(-1, keepdims=True)
    v = ((x - m) ** 2).mean(-1, keepdims=True)
    return (x - m) / jnp.sqrt(v + 1e-5) * g + b


def _col128(row):
    ii = lax.broadcasted_iota(jnp.int32, (BLK, BLK), 0)
    jj = lax.broadcasted_iota(jnp.int32, (BLK, BLK), 1)
    m = jnp.where(ii == jj, jnp.broadcast_to(row, (BLK, BLK)), 0.0)
    return jnp.sum(m, axis=1, keepdims=True)


def _k0_body(pos_r, dat_r, out_o):
    pos = pos_r[0]
    dt = dat_r[0]
    for kb in range(NBLK):
        coli = lax.broadcasted_iota(jnp.int32, (BLK, S), 0) + kb * BLK
        oh = jnp.where(coli == jnp.broadcast_to(pos, (BLK, S)), 1.0, 0.0)
        res = lax.dot_general(dt, oh, (((1,), (1,)), ((), ())),
                              preferred_element_type=jnp.float32,
                              precision=lax.Precision.HIGHEST)
        out_o[0, :, kb * BLK:(kb + 1) * BLK] = res


def _copy_body(i_ref, o_ref):
    o_ref[...] = i_ref[...]


def _linearize(x):
    return pl.pallas_call(
        _copy_body,
        out_shape=jax.ShapeDtypeStruct(x.shape, x.dtype),
    )(x)


def _sc_gather_rows(table, idx):
    n = idx.shape[0]
    d = table.shape[1]
    info = plsc.get_sparse_core_info()
    nw = info.num_cores * info.num_subcores
    rpw = n // nw
    ch = min(rpw, 64)
    nch = rpw // ch
    nbuf = min(7, nch)
    mesh = plsc.VectorSubcoreMesh(core_axis_name="c", subcore_axis_name="s")
    idx3 = idx.reshape(nw, nch, ch)

    @functools.partial(
        pl.kernel, mesh=mesh,
        out_type=jax.ShapeDtypeStruct((n, d), jnp.float32),
        scratch_types=(
            [pltpu.VMEM((nch, ch), jnp.int32)]
            + [pltpu.VMEM((ch, d), jnp.float32) for _ in range(nbuf)]
            + [pltpu.SemaphoreType.DMA for _ in range(2 * nbuf)]
        ),
    )
    def k(tab_hbm, idx_hbm, out_hbm, idx_v, *rest):
        bufs = rest[:nbuf]
        gsems = rest[nbuf:2 * nbuf]
        ssems = rest[2 * nbuf:3 * nbuf]
        wid = lax.axis_index("s") * info.num_cores + lax.axis_index("c")
        base = wid * rpw
        pltpu.sync_copy(idx_hbm.at[wid], idx_v)
        gcp = {}
        scp = {}
        for ci in range(min(nbuf, nch)):
            gcp[ci] = pltpu.async_copy(tab_hbm.at[idx_v.at[ci]],
                                       bufs[ci % nbuf], gsems[ci % nbuf])
        for ci in range(nch):
            gcp[ci].wait()
            scp[ci] = pltpu.async_copy(
                bufs[ci % nbuf],
                out_hbm.at[pl.ds(base + ci * ch, ch)],
                ssems[ci % nbuf])
            nxt = ci + nbuf
            if nxt < nch:
                scp[ci].wait()
                gcp[nxt] = pltpu.async_copy(tab_hbm.at[idx_v.at[nxt]],
                                            bufs[nxt % nbuf], gsems[nxt % nbuf])
        for ci in range(max(0, nch - nbuf), nch):
            scp[ci].wait()

    return k(table, idx3)


def _k2_body(km, tm, vld, catp, f2, f3, pr, tmr, embcat, embev, aff,
             encW, encb, encg, encbe, wq, wk, wv,
             es_o, q_o, k_o, v_o):
    b = pl.program_id(0)
    kk = pl.program_id(1)

    @pl.when(vld[b, kk] == 1)
    def _():
        ccol = _col128(catp[0])
        lanef = lax.broadcasted_iota(jnp.int32, (BLK, 1024), 1).astype(jnp.float32)
        ohc = jnp.where(lanef == ccol, 1.0, 0.0)
        fcat = jnp.dot(ohc, embcat[...], preferred_element_type=jnp.float32,
                       precision=lax.Precision.HIGHEST)
        x = fcat + f2[0] + f3[0]
        x = x + embev[0]
        pc = _col128(pr[0])
        tc = _col128(tmr[0])
        x = x + pc * aff[0:1, :] + aff[1:2, :] + tc * aff[2:3, :] + aff[3:4, :]
        h = jnp.dot(x, encW[0], preferred_element_type=jnp.float32) + encb[0]
        h = jnp.maximum(_ln(h, encg[0], encbe[0]), 0.0)
        es_o[0] = h
        q_o[0] = jnp.dot(h, wq[0], preferred_element_type=jnp.float32)
        k_o[0] = jnp.dot(h, wk[0], preferred_element_type=jnp.float32)
        v_o[0] = jnp.dot(h, wv[0], preferred_element_type=jnp.float32)


def _k3_body(km, tm, bom, nkvm, slm, q_r, kf, vf, es_r, wo, g1, b1,
             x1_o, s_ref):
    b = pl.program_id(0)
    kk = pl.program_id(1)

    @pl.when(nkvm[b, kk] > 0)
    def _():
        bo = bom[b, kk]
        nkv = nkvm[b, kk]
        sl = slm[b, kk]
        q = q_r[0]
        kiota = lax.broadcasted_iota(jnp.int32, (BLK, BLK), 1)

        def score_body(j, _):
            kb = kf[0, pl.ds((bo + j) * BLK, BLK), :]
            kvvalid = (j * BLK + kiota) < sl
            for h in range(NH):
                qh = q[:, h * DH:(h + 1) * DH]
                kh = kb[:, h * DH:(h + 1) * DH]
                s = lax.dot_general(qh, kh, (((1,), (1,)), ((), ())),
                                    preferred_element_type=jnp.float32) * SCALE
                s_ref[h, j] = jnp.where(kvvalid, s, NEG)
            return 0

        lax.fori_loop(0, nkv, score_body, 0)

        outs = []
        for h in range(NH):
            def maxb(j, m):
                return jnp.maximum(m, jnp.max(s_ref[h, j], axis=1, keepdims=True))
            m = lax.fori_loop(0, nkv, maxb, jnp.full((BLK, 1), NEG, jnp.float32))

            def pdv(j, carry):
                den, o = carry
                pj = jnp.exp(s_ref[h, j] - m)
                den = den + jnp.sum(pj, axis=1, keepdims=True)
                vb = vf[0, pl.ds((bo + j) * BLK, BLK), h * DH:(h + 1) * DH]
                o = o + jnp.dot(pj, vb, preferred_element_type=jnp.float32)
                return den, o

            den, o = lax.fori_loop(
                0, nkv, pdv,
                (jnp.zeros((BLK, 1), jnp.float32),
                 jnp.zeros((BLK, DH), jnp.float32)))
            outs.append(o / den)

        attn = jnp.concatenate(outs, axis=1)
        o = jnp.dot(attn, wo[0], preferred_element_type=jnp.float32)
        x = es_r[0] + o
        x1_o[0] = _ln(x, g1[0], b1[0])


def _k4_body(km, tm, vld, vlm, fstm, x1_r, w1, bb1, w2, bb2, g2, be2,
             x2_o, pool_o):
    b = pl.program_id(0)
    kk = pl.program_id(1)

    @pl.when(vld[b, kk] == 1)
    def _():
        x1 = x1_r[0]
        h = jnp.dot(x1, w1[0], preferred_element_type=jnp.float32)
        h = jnp.maximum(h + bb1[0], 0.0)
        y = jnp.dot(h, w2[0], preferred_element_type=jnp.float32)
        y = y + bb2[0]
        x2 = _ln(x1 + y, g2[0], be2[0])
        x2_o[0] = x2
        riota = lax.broadcasted_iota(jnp.int32, (BLK, 1), 0)
        msk = (riota < vlm[b, kk]).astype(jnp.float32)
        ps = jnp.sum(x2 * msk, axis=0, keepdims=True)

        @pl.when(fstm[b, kk] == 1)
        def _():
            pool_o[0] = ps

        @pl.when(fstm[b, kk] == 0)
        def _():
            pool_o[0] = pool_o[0] + ps


def _logsig(x):
    return jnp.minimum(x, 0.0) - jnp.log(1.0 + jnp.exp(-jnp.abs(x)))


def _k6_body(pool, cntr, fw1, fb1, fg1, fbe1, fw2, fb2, fg2, fbe2,
             wcr, bcb, wcat, bcat, wprod, bprod, churn_b, catp, prodp,
             user_o, chl_o, cat_o, prod_o, scal_o):
    u = jnp.where(cntr[...] > 0, pool[...] / jnp.maximum(cntr[...], 1.0), 0.0)
    h = jnp.dot(u, fw1[...], preferred_element_type=jnp.float32) + fb1[...]
    h = jnp.maximum(_ln(h, fg1[...], fbe1[...]), 0.0)
    us = jnp.dot(h, fw2[...], preferred_element_type=jnp.float32) + fb2[...]
    us = jnp.tanh(_ln(us, fg2[...], fbe2[...]))
    user_o[...] = us

    chl = jnp.sum(us * wcr[...], axis=1, keepdims=True) + bcb[0:1, 0:1]
    lanes128 = lax.broadcasted_iota(jnp.int32, (B, BLK), 1)
    chl_o[...] = jnp.where(lanes128 == 0, chl, 0.0)

    cat = jnp.dot(us, wcat[...], preferred_element_type=jnp.float32) + bcat[...]
    cat_o[...] = cat
    prod = jnp.dot(us, wprod[...], preferred_element_type=jnp.float32) + bprod[...]
    prod_o[...] = prod

    churn_col = churn_b[...][:, 0:1]
    pw = jnp.where(jnp.sum(churn_col) > 0.0, 5.0, 1.0)
    tch = -(pw * churn_col * _logsig(chl) + (1.0 - churn_col) * _logsig(-chl))
    cl = jnp.sum(tch) / B

    ycat = (catp[...] > 0.0).astype(jnp.float32)
    mcat = (lanes128 < NCAT).astype(jnp.float32)
    tcat = -(ycat * _logsig(cat) + (1.0 - ycat) * _logsig(-cat)) * mcat
    catl = jnp.sum(tcat) / (B * NCAT)

    lanes1024 = lax.broadcasted_iota(jnp.int32, (B, 1024), 1)
    yprod = (prodp[...] > 0.0).astype(jnp.float32)
    mprod = (lanes1024 < NPROD).astype(jnp.float32)
    tprod = -(yprod * _logsig(prod) + (1.0 - yprod) * _logsig(-prod)) * mprod
    prodl = jnp.sum(tprod) / (B * NPROD)

    total = cl + 0.4 * catl + 0.4 * prodl
    total = jnp.where(jnp.isnan(total) | jnp.isinf(total), 100.0, total)
    slanes = lax.broadcasted_iota(jnp.int32, (1, BLK), 1)
    sc = jnp.where(slanes == 0, cl, 0.0)
    sc = jnp.where(slanes == 1, catl, sc)
    sc = jnp.where(slanes == 2, prodl, sc)
    sc = jnp.where(slanes == 3, total * 0.1, sc)
    scal_o[...] = sc


def kernel(event_types, categories, prices, names, queries, timestamps,
           mask, churn, category_propensity, product_propensity,
           client_id, params):
    p = params
    et = event_types.astype(jnp.int32)

    onehotf = (et[:, :, None] == jnp.arange(5)[None, None, :]).astype(jnp.float32)
    c = onehotf.sum(1).astype(jnp.int32)
    csum = jnp.cumsum(onehotf, axis=1)
    rank = (onehotf * csum).sum(-1) - 1.0
    nb = (c + BLK - 1) // BLK
    blk_end = jnp.cumsum(nb, axis=1)
    blk_off = blk_end - nb
    nblk = blk_end[:, -1]
    bo_tok = (onehotf * blk_off[:, None, :].astype(jnp.float32)).sum(-1)
    pos_map_f = bo_tok * BLK + rank
    pos_map = pos_map_f.astype(jnp.int32)

    kk = jnp.arange(NBLK)[None, :]
    k_eff = jnp.minimum(kk, (nblk - 1)[:, None])
    t_of = (k_eff[:, :, None] >= blk_end[:, None, :]).sum(-1).astype(jnp.int32)
    toh = (t_of[:, :, None] == jnp.arange(5)[None, None, :]).astype(jnp.int32)
    bo = (toh * blk_off[:, None, :]).sum(-1)
    nkv = (toh * nb[:, None, :]).sum(-1)
    seg_len = (toh * c[:, None, :]).sum(-1)
    blk_in_seg = k_eff - bo
    vlen = jnp.clip(seg_len - blk_in_seg * BLK, 0, BLK)
    validb = (kk < nblk[:, None]).astype(jnp.int32)
    firstb = ((blk_in_seg == 0) & (validb == 1)).astype(jnp.int32)
    nkv_g = (nkv * validb).astype(jnp.int32)

    km = k_eff.astype(jnp.int32)
    tmb = t_of
    bob = bo.astype(jnp.int32)
    slb = seg_len.astype(jnp.int32)
    vlb = vlen.astype(jnp.int32)

    dataT = jnp.stack([categories.astype(jnp.float32),
                       names.astype(jnp.float32),
                       queries.astype(jnp.float32),
                       prices, timestamps], axis=1)
    perm = pl.pallas_call(
        _k0_body,
        grid=(B,),
        in_specs=[pl.BlockSpec((1, 1, S), lambda b: (b, 0, 0)),
                  pl.BlockSpec((1, 5, S), lambda b: (b, 0, 0))],
        out_specs=pl.BlockSpec((1, 5, P), lambda b: (b, 0, 0)),
        out_shape=jax.ShapeDtypeStruct((B, 5, P), jnp.float32),
    )(pos_map.reshape(B, 1, S), dataT)
    cat_p = jnp.round(perm[:, 0]).astype(jnp.int32)
    name_p = jnp.round(perm[:, 1]).astype(jnp.int32)
    query_p = jnp.round(perm[:, 2]).astype(jnp.int32)
    price_p = perm[:, 3]
    time_p = perm[:, 4]

    f_name = _sc_gather_rows(p['emb_name'],
                             name_p.reshape(-1)).reshape(B, P, H)
    f_query = _sc_gather_rows(p['emb_query'],
                              query_p.reshape(-1)).reshape(B, P, H)
    embcat_pad = jnp.pad(p['emb_cat'], ((0, 1024 - 1000), (0, 0)))
    catp3 = perm[:, 0].reshape(B * NBLK, 1, BLK)

    aff = jnp.stack([p['w_price'], p['b_price'], p['w_time'], p['b_time']], 0)
    pr3 = price_p.reshape(B * NBLK, 1, BLK)
    tm3 = time_p.reshape(B * NBLK, 1, BLK)

    tok_spec = pl.BlockSpec((1, BLK, H), lambda b, k, km, tm, vd: (b, km[b, k], 0))
    sc_spec = pl.BlockSpec((1, 1, BLK),
                           lambda b, k, km, tm, vd: (b * NBLK + km[b, k], 0, 0))
    whh = pl.BlockSpec((1, H, H), lambda b, k, km, tm, vd: (tm[b, k], 0, 0))
    wh = pl.BlockSpec((1, 1, H), lambda b, k, km, tm, vd: (tm[b, k], 0, 0))

    es, q, k_, v = pl.pallas_call(
        _k2_body,
        grid_spec=pltpu.PrefetchScalarGridSpec(
            num_scalar_prefetch=3,
            grid=(B, NBLK),
            in_specs=[
                sc_spec, tok_spec, tok_spec, sc_spec, sc_spec,
                pl.BlockSpec((1024, H), lambda b, k, *r: (0, 0)),
                wh,
                pl.BlockSpec((4, H), lambda b, k, *r: (0, 0)),
                whh, wh, wh, wh,
                whh, whh, whh,
            ],
            out_specs=[tok_spec, tok_spec, tok_spec, tok_spec],
        ),
        out_shape=[jax.ShapeDtypeStruct((B, P, H), jnp.float32)] * 4,
    )(km, tmb, validb,
      catp3, f_name, f_query, pr3, tm3,
      embcat_pad,
      p['emb_event'].reshape(5, 1, H), aff,
      p['enc_W'], p['enc_b'].reshape(5, 1, H), p['enc_g'].reshape(5, 1, H),
      p['enc_be'].reshape(5, 1, H),
      p['Wq'], p['Wk'], p['Wv'])

    tok5 = pl.BlockSpec((1, BLK, H),
                        lambda b, k, km, tm, bo, nk, sl: (b, km[b, k], 0))
    row5 = pl.BlockSpec((1, P, H),
                        lambda b, k, km, tm, bo, nk, sl: (b, 0, 0))

    x1 = pl.pallas_call(
        _k3_body,
        grid_spec=pltpu.PrefetchScalarGridSpec(
            num_scalar_prefetch=5,
            grid=(B, NBLK),
            in_specs=[tok5, row5, row5, tok5,
                      pl.BlockSpec((1, H, H),
                                   lambda b, k, km, tm, bo, nk, sl: (tm[b, k], 0, 0)),
                      pl.BlockSpec((1, 1, H),
                                   lambda b, k, km, tm, bo, nk, sl: (tm[b, k], 0, 0)),
                      pl.BlockSpec((1, 1, H),
                                   lambda b, k, km, tm, bo, nk, sl: (tm[b, k], 0, 0))],
            out_specs=[tok5],
            scratch_shapes=[pltpu.VMEM((NH, 4, BLK, BLK), jnp.float32)],
        ),
        out_shape=[jax.ShapeDtypeStruct((B, P, H), jnp.float32)],
    )(km, tmb, bob, nkv_g, slb,
      q, k_, v, es, p['Wo'], p['ln1_g'].reshape(5, 1, H),
      p['ln1_b'].reshape(5, 1, H))[0]

    tok4 = pl.BlockSpec((1, BLK, H),
                        lambda b, k, km, tm, vd, vl, fs: (b, km[b, k], 0))
    x2, pool = pl.pallas_call(
        _k4_body,
        grid_spec=pltpu.PrefetchScalarGridSpec(
            num_scalar_prefetch=5,
            grid=(B, NBLK),
            in_specs=[
                tok4,
                pl.BlockSpec((1, H, 4 * H),
                             lambda b, k, km, tm, vd, vl, fs: (tm[b, k], 0, 0)),
                pl.BlockSpec((1, 1, 4 * H),
                             lambda b, k, km, tm, vd, vl, fs: (tm[b, k], 0, 0)),
                pl.BlockSpec((1, 4 * H, H),
                             lambda b, k, km, tm, vd, vl, fs: (tm[b, k], 0, 0)),
                pl.BlockSpec((1, 1, H),
                             lambda b, k, km, tm, vd, vl, fs: (tm[b, k], 0, 0)),
                pl.BlockSpec((1, 1, H),
                             lambda b, k, km, tm, vd, vl, fs: (tm[b, k], 0, 0)),
                pl.BlockSpec((1, 1, H),
                             lambda b, k, km, tm, vd, vl, fs: (tm[b, k], 0, 0)),
            ],
            out_specs=[
                tok4,
                pl.BlockSpec((1, 1, H),
                             lambda b, k, km, tm, vd, vl, fs: (b * 5 + tm[b, k], 0, 0)),
            ],
        ),
        out_shape=[jax.ShapeDtypeStruct((B, P, H), jnp.float32),
                   jax.ShapeDtypeStruct((B * 5, 1, H), jnp.float32)],
    )(km, tmb, validb, vlb, firstb,
      x1, p['f_W1'], p['f_b1'].reshape(5, 1, 4 * H), p['f_W2'],
      p['f_b2'].reshape(5, 1, H), p['ln2_g'].reshape(5, 1, H),
      p['ln2_b'].reshape(5, 1, H))

    tidx = (jnp.arange(B)[:, None] * P + pos_map).reshape(-1).astype(jnp.int32)
    temporal = _sc_gather_rows(x2.reshape(B * P, H), tidx).reshape(B, S, H)

    pool2 = pool.reshape(B, 5 * H)
    cntr = jnp.repeat(c.astype(jnp.float32), H, axis=1)
    churn_b = jnp.broadcast_to(churn[:, None], (B, BLK))
    catp = jnp.pad(category_propensity, ((0, 0), (0, BLK - NCAT)),
                   constant_values=-1.0)
    prodp = jnp.pad(product_propensity, ((0, 0), (0, 1024 - NPROD)),
                    constant_values=-1.0)
    wcat = jnp.pad(p['Wcat'], ((0, 0), (0, BLK - NCAT)))
    bcat = jnp.pad(p['bcat'], (0, BLK - NCAT)).reshape(1, BLK)
    wprod = jnp.pad(p['Wprod'], ((0, 0), (0, 1024 - NPROD)))
    bprod = jnp.pad(p['bprod'], (0, 1024 - NPROD)).reshape(1, 1024)
    wcr = p['Wc'].reshape(1, H)
    bcb = jnp.broadcast_to(p['bc'].reshape(1, 1), (1, BLK))

    user, chl_o, cat_o, prod_o, scal = pl.pallas_call(
        _k6_body,
        out_shape=[
            jax.ShapeDtypeStruct((B, H), jnp.float32),
            jax.ShapeDtypeStruct((B, BLK), jnp.float32),
            jax.ShapeDtypeStruct((B, BLK), jnp.float32),
            jax.ShapeDtypeStruct((B, 1024), jnp.float32),
            jax.ShapeDtypeStruct((1, BLK), jnp.float32),
        ],
    )(pool2, cntr,
      p['fus_W1'], p['fus_b1'].reshape(1, 2 * H), p['fus_g1'].reshape(1, 2 * H),
      p['fus_be1'].reshape(1, 2 * H),
      p['fus_W2'], p['fus_b2'].reshape(1, H), p['fus_g2'].reshape(1, H),
      p['fus_be2'].reshape(1, H),
      wcr, bcb, wcat, bcat, wprod, bprod, churn_b, catp, prodp)

    churn_logits = chl_o[:, 0]
    cat_logits = cat_o[:, :NCAT]
    prod_logits = prod_o[:, :NPROD]
    return (user, temporal, churn_logits, cat_logits, prod_logits,
            scal[0, 0], scal[0, 1], scal[0, 2], scal[0, 3])


# speedup vs baseline: 1.1317x; 1.1317x over previous
import functools

import jax
import jax.numpy as jnp
from jax import lax
from jax.experimental import pallas as pl
from jax.experimental.pallas import tpu as pltpu
from jax.experimental.pallas import tpu_sc as plsc

B, S, H, NH, DH = 16, 512, 256, 4, 64
NCAT, NPROD = 100, 1000
BLK = 128
NBLK = 8
P = NBLK * BLK
NEG = -1e9
SCALE = 1.0 / (DH ** 0.5)


def _ln(x, g, b):
    m = x.mean(-1, keepdims=True)
    v = ((x - m) ** 2).mean(-1, keepdims=True)
    return (x - m) / jnp.sqrt(v + 1e-5) * g + b


def _col128(row):
    ii = lax.broadcasted_iota(jnp.int32, (BLK, BLK), 0)
    jj = lax.broadcasted_iota(jnp.int32, (BLK, BLK), 1)
    m = jnp.where(ii == jj, jnp.broadcast_to(row, (BLK, BLK)), 0.0)
    return jnp.sum(m, axis=1, keepdims=True)


def _k0_body(pos_r, dat_r, out_o):
    pos = pos_r[0]
    dt = dat_r[0]
    for kb in range(NBLK):
        coli = lax.broadcasted_iota(jnp.int32, (BLK, S), 0) + kb * BLK
        oh = jnp.where(coli == jnp.broadcast_to(pos, (BLK, S)), 1.0, 0.0)
        res = lax.dot_general(dt, oh, (((1,), (1,)), ((), ())),
                              preferred_element_type=jnp.float32,
                              precision=lax.Precision.HIGHEST)
        out_o[0, :, kb * BLK:(kb + 1) * BLK] = res


def _copy_body(i_ref, o_ref):
    o_ref[...] = i_ref[...]


def _linearize(x):
    return pl.pallas_call(
        _copy_body,
        out_shape=jax.ShapeDtypeStruct(x.shape, x.dtype),
    )(x)


def _sc_gather_rows(table, idx):
    n = idx.shape[0]
    d = table.shape[1]
    info = plsc.get_sparse_core_info()
    nw = info.num_cores * info.num_subcores
    rpw = n // nw
    ch = min(rpw, 64)
    nch = rpw // ch
    nbuf = min(7, nch)
    mesh = plsc.VectorSubcoreMesh(core_axis_name="c", subcore_axis_name="s")
    idx3 = idx.reshape(nw, nch, ch)

    @functools.partial(
        pl.kernel, mesh=mesh,
        out_type=jax.ShapeDtypeStruct((n, d), jnp.float32),
        scratch_types=(
            [pltpu.VMEM((nch, ch), jnp.int32)]
            + [pltpu.VMEM((ch, d), jnp.float32) for _ in range(nbuf)]
            + [pltpu.SemaphoreType.DMA for _ in range(2 * nbuf)]
        ),
    )
    def k(tab_hbm, idx_hbm, out_hbm, idx_v, *rest):
        bufs = rest[:nbuf]
        gsems = rest[nbuf:2 * nbuf]
        ssems = rest[2 * nbuf:3 * nbuf]
        wid = lax.axis_index("s") * info.num_cores + lax.axis_index("c")
        base = wid * rpw
        pltpu.sync_copy(idx_hbm.at[wid], idx_v)
        gcp = {}
        scp = {}
        for ci in range(min(nbuf, nch)):
            gcp[ci] = pltpu.async_copy(tab_hbm.at[idx_v.at[ci]],
                                       bufs[ci % nbuf], gsems[ci % nbuf])
        for ci in range(nch):
            gcp[ci].wait()
            scp[ci] = pltpu.async_copy(
                bufs[ci % nbuf],
                out_hbm.at[pl.ds(base + ci * ch, ch)],
                ssems[ci % nbuf])
            nxt = ci + nbuf
            if nxt < nch:
                scp[ci].wait()
                gcp[nxt] = pltpu.async_copy(tab_hbm.at[idx_v.at[nxt]],
                                            bufs[nxt % nbuf], gsems[nxt % nbuf])
        for ci in range(max(0, nch - nbuf), nch):
            scp[ci].wait()

    return k(table, idx3)


def _k2_body(km, tm, vld, catp, f2, f3, pr, tmr, embcat, embev, aff,
             encW, encb, encg, encbe, wq, wk, wv,
             es_o, q_o, k_o, v_o):
    b = pl.program_id(0)
    kk = pl.program_id(1)

    @pl.when(vld[b, kk] == 1)
    def _():
        ccol = _col128(catp[0])
        lanef = lax.broadcasted_iota(jnp.int32, (BLK, 1024), 1).astype(jnp.float32)
        ohc = jnp.where(lanef == ccol, 1.0, 0.0)
        fcat = jnp.dot(ohc, embcat[...], preferred_element_type=jnp.float32,
                       precision=lax.Precision.HIGHEST)
        x = fcat + f2[0] + f3[0]
        x = x + embev[0]
        pc = _col128(pr[0])
        tc = _col128(tmr[0])
        x = x + pc * aff[0:1, :] + aff[1:2, :] + tc * aff[2:3, :] + aff[3:4, :]
        h = jnp.dot(x, encW[0], preferred_element_type=jnp.float32) + encb[0]
        h = jnp.maximum(_ln(h, encg[0], encbe[0]), 0.0)
        es_o[0] = h
        q_o[0] = jnp.dot(h, wq[0], preferred_element_type=jnp.float32)
        k_o[0] = jnp.dot(h, wk[0], preferred_element_type=jnp.float32)
        v_o[0] = jnp.dot(h, wv[0], preferred_element_type=jnp.float32)


def _k3_body(km, tm, bom, nkvm, slm, q_r, kf, vf, es_r, wo, g1, b1,
             x1_o, s_ref):
    b = pl.program_id(0)
    kk = pl.program_id(1)

    @pl.when(nkvm[b, kk] > 0)
    def _():
        bo = bom[b, kk]
        nkv = nkvm[b, kk]
        sl = slm[b, kk]
        q = q_r[0]
        kiota = lax.broadcasted_iota(jnp.int32, (BLK, BLK), 1)

        def score_body(j, _):
            kb = kf[0, pl.ds((bo + j) * BLK, BLK), :]
            kvvalid = (j * BLK + kiota) < sl
            for h in range(NH):
                qh = q[:, h * DH:(h + 1) * DH]
                kh = kb[:, h * DH:(h + 1) * DH]
                s = lax.dot_general(qh, kh, (((1,), (1,)), ((), ())),
                                    preferred_element_type=jnp.float32) * SCALE
                s_ref[h, j] = jnp.where(kvvalid, s, NEG)
            return 0

        lax.fori_loop(0, nkv, score_body, 0)

        outs = []
        for h in range(NH):
            def maxb(j, m):
                return jnp.maximum(m, jnp.max(s_ref[h, j], axis=1, keepdims=True))
            m = lax.fori_loop(0, nkv, maxb, jnp.full((BLK, 1), NEG, jnp.float32))

            def pdv(j, carry):
                den, o = carry
                pj = jnp.exp(s_ref[h, j] - m)
                den = den + jnp.sum(pj, axis=1, keepdims=True)
                vb = vf[0, pl.ds((bo + j) * BLK, BLK), h * DH:(h + 1) * DH]
                o = o + jnp.dot(pj, vb, preferred_element_type=jnp.float32)
                return den, o

            den, o = lax.fori_loop(
                0, nkv, pdv,
                (jnp.zeros((BLK, 1), jnp.float32),
                 jnp.zeros((BLK, DH), jnp.float32)))
            outs.append(o / den)

        attn = jnp.concatenate(outs, axis=1)
        o = jnp.dot(attn, wo[0], preferred_element_type=jnp.float32)
        x = es_r[0] + o
        x1_o[0] = _ln(x, g1[0], b1[0])


def _k4_body(km, tm, vld, vlm, fstm, x1_r, w1, bb1, w2, bb2, g2, be2,
             x2_o, pool_o):
    b = pl.program_id(0)
    kk = pl.program_id(1)

    @pl.when(vld[b, kk] == 1)
    def _():
        x1 = x1_r[0]
        h = jnp.dot(x1, w1[0], preferred_element_type=jnp.float32)
        h = jnp.maximum(h + bb1[0], 0.0)
        y = jnp.dot(h, w2[0], preferred_element_type=jnp.float32)
        y = y + bb2[0]
        x2 = _ln(x1 + y, g2[0], be2[0])
        x2_o[0] = x2
        riota = lax.broadcasted_iota(jnp.int32, (BLK, 1), 0)
        msk = (riota < vlm[b, kk]).astype(jnp.float32)
        ps = jnp.sum(x2 * msk, axis=0, keepdims=True)

        @pl.when(fstm[b, kk] == 1)
        def _():
            pool_o[0] = ps

        @pl.when(fstm[b, kk] == 0)
        def _():
            pool_o[0] = pool_o[0] + ps


def _logsig(x):
    return jnp.minimum(x, 0.0) - jnp.log(1.0 + jnp.exp(-jnp.abs(x)))


def _k6_body(pool, cntr, fw1, fb1, fg1, fbe1, fw2, fb2, fg2, fbe2,
             wcr, bcb, wcat, bcat, wprod, bprod, churn_b, catp, prodp,
             user_o, chl_o, cat_o, prod_o, scal_o):
    u = jnp.where(cntr[...] > 0, pool[...] / jnp.maximum(cntr[...], 1.0), 0.0)
    h = jnp.dot(u, fw1[...], preferred_element_type=jnp.float32) + fb1[...]
    h = jnp.maximum(_ln(h, fg1[...], fbe1[...]), 0.0)
    us = jnp.dot(h, fw2[...], preferred_element_type=jnp.float32) + fb2[...]
    us = jnp.tanh(_ln(us, fg2[...], fbe2[...]))
    user_o[...] = us

    chl = jnp.sum(us * wcr[...], axis=1, keepdims=True) + bcb[0:1, 0:1]
    lanes128 = lax.broadcasted_iota(jnp.int32, (B, BLK), 1)
    chl_o[...] = jnp.where(lanes128 == 0, chl, 0.0)

    cat = jnp.dot(us, wcat[...], preferred_element_type=jnp.float32) + bcat[...]
    cat_o[...] = cat
    prod = jnp.dot(us, wprod[...], preferred_element_type=jnp.float32) + bprod[...]
    prod_o[...] = prod

    churn_col = churn_b[...][:, 0:1]
    pw = jnp.where(jnp.sum(churn_col) > 0.0, 5.0, 1.0)
    tch = -(pw * churn_col * _logsig(chl) + (1.0 - churn_col) * _logsig(-chl))
    cl = jnp.sum(tch) / B

    ycat = (catp[...] > 0.0).astype(jnp.float32)
    mcat = (lanes128 < NCAT).astype(jnp.float32)
    tcat = -(ycat * _logsig(cat) + (1.0 - ycat) * _logsig(-cat)) * mcat
    catl = jnp.sum(tcat) / (B * NCAT)

    lanes1024 = lax.broadcasted_iota(jnp.int32, (B, 1024), 1)
    yprod = (prodp[...] > 0.0).astype(jnp.float32)
    mprod = (lanes1024 < NPROD).astype(jnp.float32)
    tprod = -(yprod * _logsig(prod) + (1.0 - yprod) * _logsig(-prod)) * mprod
    prodl = jnp.sum(tprod) / (B * NPROD)

    total = cl + 0.4 * catl + 0.4 * prodl
    total = jnp.where(jnp.isnan(total) | jnp.isinf(total), 100.0, total)
    slanes = lax.broadcasted_iota(jnp.int32, (1, BLK), 1)
    sc = jnp.where(slanes == 0, cl, 0.0)
    sc = jnp.where(slanes == 1, catl, sc)
    sc = jnp.where(slanes == 2, prodl, sc)
    sc = jnp.where(slanes == 3, total * 0.1, sc)
    scal_o[...] = sc


def _encode_half(et, categories, names, queries, prices, timestamps, p):
    B = et.shape[0]
    onehotf = (et[:, :, None] == jnp.arange(5)[None, None, :]).astype(jnp.float32)
    c = onehotf.sum(1).astype(jnp.int32)
    csum = jnp.cumsum(onehotf, axis=1)
    rank = (onehotf * csum).sum(-1) - 1.0
    nb = (c + BLK - 1) // BLK
    blk_end = jnp.cumsum(nb, axis=1)
    blk_off = blk_end - nb
    nblk = blk_end[:, -1]
    bo_tok = (onehotf * blk_off[:, None, :].astype(jnp.float32)).sum(-1)
    pos_map_f = bo_tok * BLK + rank
    pos_map = pos_map_f.astype(jnp.int32)

    kk = jnp.arange(NBLK)[None, :]
    k_eff = jnp.minimum(kk, (nblk - 1)[:, None])
    t_of = (k_eff[:, :, None] >= blk_end[:, None, :]).sum(-1).astype(jnp.int32)
    toh = (t_of[:, :, None] == jnp.arange(5)[None, None, :]).astype(jnp.int32)
    bo = (toh * blk_off[:, None, :]).sum(-1)
    nkv = (toh * nb[:, None, :]).sum(-1)
    seg_len = (toh * c[:, None, :]).sum(-1)
    blk_in_seg = k_eff - bo
    vlen = jnp.clip(seg_len - blk_in_seg * BLK, 0, BLK)
    validb = (kk < nblk[:, None]).astype(jnp.int32)
    firstb = ((blk_in_seg == 0) & (validb == 1)).astype(jnp.int32)
    nkv_g = (nkv * validb).astype(jnp.int32)

    km = k_eff.astype(jnp.int32)
    tmb = t_of
    bob = bo.astype(jnp.int32)
    slb = seg_len.astype(jnp.int32)
    vlb = vlen.astype(jnp.int32)

    dataT = jnp.stack([categories.astype(jnp.float32),
                       names.astype(jnp.float32),
                       queries.astype(jnp.float32),
                       prices, timestamps], axis=1)
    perm = pl.pallas_call(
        _k0_body,
        grid=(B,),
        in_specs=[pl.BlockSpec((1, 1, S), lambda b: (b, 0, 0)),
                  pl.BlockSpec((1, 5, S), lambda b: (b, 0, 0))],
        out_specs=pl.BlockSpec((1, 5, P), lambda b: (b, 0, 0)),
        out_shape=jax.ShapeDtypeStruct((B, 5, P), jnp.float32),
    )(pos_map.reshape(B, 1, S), dataT)
    cat_p = jnp.round(perm[:, 0]).astype(jnp.int32)
    name_p = jnp.round(perm[:, 1]).astype(jnp.int32)
    query_p = jnp.round(perm[:, 2]).astype(jnp.int32)
    price_p = perm[:, 3]
    time_p = perm[:, 4]

    f_name = _sc_gather_rows(p['emb_name'],
                             name_p.reshape(-1)).reshape(B, P, H)
    f_query = _sc_gather_rows(p['emb_query'],
                              query_p.reshape(-1)).reshape(B, P, H)
    embcat_pad = jnp.pad(p['emb_cat'], ((0, 1024 - 1000), (0, 0)))
    catp3 = perm[:, 0].reshape(B * NBLK, 1, BLK)

    aff = jnp.stack([p['w_price'], p['b_price'], p['w_time'], p['b_time']], 0)
    pr3 = price_p.reshape(B * NBLK, 1, BLK)
    tm3 = time_p.reshape(B * NBLK, 1, BLK)

    tok_spec = pl.BlockSpec((1, BLK, H), lambda b, k, km, tm, vd: (b, km[b, k], 0))
    sc_spec = pl.BlockSpec((1, 1, BLK),
                           lambda b, k, km, tm, vd: (b * NBLK + km[b, k], 0, 0))
    whh = pl.BlockSpec((1, H, H), lambda b, k, km, tm, vd: (tm[b, k], 0, 0))
    wh = pl.BlockSpec((1, 1, H), lambda b, k, km, tm, vd: (tm[b, k], 0, 0))

    es, q, k_, v = pl.pallas_call(
        _k2_body,
        grid_spec=pltpu.PrefetchScalarGridSpec(
            num_scalar_prefetch=3,
            grid=(B, NBLK),
            in_specs=[
                sc_spec, tok_spec, tok_spec, sc_spec, sc_spec,
                pl.BlockSpec((1024, H), lambda b, k, *r: (0, 0)),
                wh,
                pl.BlockSpec((4, H), lambda b, k, *r: (0, 0)),
                whh, wh, wh, wh,
                whh, whh, whh,
            ],
            out_specs=[tok_spec, tok_spec, tok_spec, tok_spec],
        ),
        out_shape=[jax.ShapeDtypeStruct((B, P, H), jnp.float32)] * 4,
    )(km, tmb, validb,
      catp3, f_name, f_query, pr3, tm3,
      embcat_pad,
      p['emb_event'].reshape(5, 1, H), aff,
      p['enc_W'], p['enc_b'].reshape(5, 1, H), p['enc_g'].reshape(5, 1, H),
      p['enc_be'].reshape(5, 1, H),
      p['Wq'], p['Wk'], p['Wv'])

    tok5 = pl.BlockSpec((1, BLK, H),
                        lambda b, k, km, tm, bo, nk, sl: (b, km[b, k], 0))
    row5 = pl.BlockSpec((1, P, H),
                        lambda b, k, km, tm, bo, nk, sl: (b, 0, 0))

    x1 = pl.pallas_call(
        _k3_body,
        grid_spec=pltpu.PrefetchScalarGridSpec(
            num_scalar_prefetch=5,
            grid=(B, NBLK),
            in_specs=[tok5, row5, row5, tok5,
                      pl.BlockSpec((1, H, H),
                                   lambda b, k, km, tm, bo, nk, sl: (tm[b, k], 0, 0)),
                      pl.BlockSpec((1, 1, H),
                                   lambda b, k, km, tm, bo, nk, sl: (tm[b, k], 0, 0)),
                      pl.BlockSpec((1, 1, H),
                                   lambda b, k, km, tm, bo, nk, sl: (tm[b, k], 0, 0))],
            out_specs=[tok5],
            scratch_shapes=[pltpu.VMEM((NH, 4, BLK, BLK), jnp.float32)],
        ),
        out_shape=[jax.ShapeDtypeStruct((B, P, H), jnp.float32)],
    )(km, tmb, bob, nkv_g, slb,
      q, k_, v, es, p['Wo'], p['ln1_g'].reshape(5, 1, H),
      p['ln1_b'].reshape(5, 1, H))[0]

    tok4 = pl.BlockSpec((1, BLK, H),
                        lambda b, k, km, tm, vd, vl, fs: (b, km[b, k], 0))
    x2, pool = pl.pallas_call(
        _k4_body,
        grid_spec=pltpu.PrefetchScalarGridSpec(
            num_scalar_prefetch=5,
            grid=(B, NBLK),
            in_specs=[
                tok4,
                pl.BlockSpec((1, H, 4 * H),
                             lambda b, k, km, tm, vd, vl, fs: (tm[b, k], 0, 0)),
                pl.BlockSpec((1, 1, 4 * H),
                             lambda b, k, km, tm, vd, vl, fs: (tm[b, k], 0, 0)),
                pl.BlockSpec((1, 4 * H, H),
                             lambda b, k, km, tm, vd, vl, fs: (tm[b, k], 0, 0)),
                pl.BlockSpec((1, 1, H),
                             lambda b, k, km, tm, vd, vl, fs: (tm[b, k], 0, 0)),
                pl.BlockSpec((1, 1, H),
                             lambda b, k, km, tm, vd, vl, fs: (tm[b, k], 0, 0)),
                pl.BlockSpec((1, 1, H),
                             lambda b, k, km, tm, vd, vl, fs: (tm[b, k], 0, 0)),
            ],
            out_specs=[
                tok4,
                pl.BlockSpec((1, 1, H),
                             lambda b, k, km, tm, vd, vl, fs: (b * 5 + tm[b, k], 0, 0)),
            ],
        ),
        out_shape=[jax.ShapeDtypeStruct((B, P, H), jnp.float32),
                   jax.ShapeDtypeStruct((B * 5, 1, H), jnp.float32)],
    )(km, tmb, validb, vlb, firstb,
      x1, p['f_W1'], p['f_b1'].reshape(5, 1, 4 * H), p['f_W2'],
      p['f_b2'].reshape(5, 1, H), p['ln2_g'].reshape(5, 1, H),
      p['ln2_b'].reshape(5, 1, H))

    tidx = (jnp.arange(B)[:, None] * P + pos_map).reshape(-1).astype(jnp.int32)
    temporal = _sc_gather_rows(x2.reshape(B * P, H), tidx).reshape(B, S, H)

    return pool, c, temporal


def kernel(event_types, categories, prices, names, queries, timestamps,
           mask, churn, category_propensity, product_propensity,
           client_id, params):
    p = params
    et = event_types.astype(jnp.int32)
    HB = 8
    pool_a, c_a, temp_a = _encode_half(
        et[:HB], categories[:HB], names[:HB], queries[:HB],
        prices[:HB], timestamps[:HB], p)
    pool_b, c_b, temp_b = _encode_half(
        et[HB:], categories[HB:], names[HB:], queries[HB:],
        prices[HB:], timestamps[HB:], p)
    B = 16
    pool = jnp.concatenate([pool_a, pool_b], axis=0)
    c = jnp.concatenate([c_a, c_b], axis=0)
    temporal = jnp.concatenate([temp_a, temp_b], axis=0)

    pool2 = pool.reshape(B, 5 * H)
    cntr = jnp.repeat(c.astype(jnp.float32), H, axis=1)
    churn_b = jnp.broadcast_to(churn[:, None], (B, BLK))
    catp = jnp.pad(category_propensity, ((0, 0), (0, BLK - NCAT)),
                   constant_values=-1.0)
    prodp = jnp.pad(product_propensity, ((0, 0), (0, 1024 - NPROD)),
                    constant_values=-1.0)
    wcat = jnp.pad(p['Wcat'], ((0, 0), (0, BLK - NCAT)))
    bcat = jnp.pad(p['bcat'], (0, BLK - NCAT)).reshape(1, BLK)
    wprod = jnp.pad(p['Wprod'], ((0, 0), (0, 1024 - NPROD)))
    bprod = jnp.pad(p['bprod'], (0, 1024 - NPROD)).reshape(1, 1024)
    wcr = p['Wc'].reshape(1, H)
    bcb = jnp.broadcast_to(p['bc'].reshape(1, 1), (1, BLK))

    user, chl_o, cat_o, prod_o, scal = pl.pallas_call(
        _k6_body,
        out_shape=[
            jax.ShapeDtypeStruct((B, H), jnp.float32),
            jax.ShapeDtypeStruct((B, BLK), jnp.float32),
            jax.ShapeDtypeStruct((B, BLK), jnp.float32),
            jax.ShapeDtypeStruct((B, 1024), jnp.float32),
            jax.ShapeDtypeStruct((1, BLK), jnp.float32),
        ],
    )(pool2, cntr,
      p['fus_W1'], p['fus_b1'].reshape(1, 2 * H), p['fus_g1'].reshape(1, 2 * H),
      p['fus_be1'].reshape(1, 2 * H),
      p['fus_W2'], p['fus_b2'].reshape(1, H), p['fus_g2'].reshape(1, H),
      p['fus_be2'].reshape(1, H),
      wcr, bcb, wcat, bcat, wprod, bprod, churn_b, catp, prodp)

    churn_logits = chl_o[:, 0]
    cat_logits = cat_o[:, :NCAT]
    prod_logits = prod_o[:, :NPROD]
    return (user, temporal, churn_logits, cat_logits, prod_logits,
            scal[0, 0], scal[0, 1], scal[0, 2], scal[0, 3])


# speedup vs baseline: 1.2002x; 1.0606x over previous
import functools

import jax
import jax.numpy as jnp
from jax import lax
from jax.experimental import pallas as pl
from jax.experimental.pallas import tpu as pltpu
from jax.experimental.pallas import tpu_sc as plsc

B, S, H, NH, DH = 16, 512, 256, 4, 64
NCAT, NPROD = 100, 1000
BLK = 128
NBLK = 8
P = NBLK * BLK
NEG = -1e9
SCALE = 1.0 / (DH ** 0.5)


def _ln(x, g, b):
    m = x.mean(-1, keepdims=True)
    v = ((x - m) ** 2).mean(-1, keepdims=True)
    return (x - m) / jnp.sqrt(v + 1e-5) * g + b


def _col128(row):
    ii = lax.broadcasted_iota(jnp.int32, (BLK, BLK), 0)
    jj = lax.broadcasted_iota(jnp.int32, (BLK, BLK), 1)
    m = jnp.where(ii == jj, jnp.broadcast_to(row, (BLK, BLK)), 0.0)
    return jnp.sum(m, axis=1, keepdims=True)


def _k0_body(pos_r, dat_r, out_o):
    pos = pos_r[0]
    dt = dat_r[0]
    for kb in range(NBLK):
        coli = lax.broadcasted_iota(jnp.int32, (BLK, S), 0) + kb * BLK
        oh = jnp.where(coli == jnp.broadcast_to(pos, (BLK, S)), 1.0, 0.0)
        res = lax.dot_general(dt, oh, (((1,), (1,)), ((), ())),
                              preferred_element_type=jnp.float32,
                              precision=lax.Precision.HIGHEST)
        out_o[0, :, kb * BLK:(kb + 1) * BLK] = res


def _copy_body(i_ref, o_ref):
    o_ref[...] = i_ref[...]


def _linearize(x):
    return pl.pallas_call(
        _copy_body,
        out_shape=jax.ShapeDtypeStruct(x.shape, x.dtype),
    )(x)


def _sc_gather_rows(table, idx):
    n = idx.shape[0]
    d = table.shape[1]
    info = plsc.get_sparse_core_info()
    nw = info.num_cores * info.num_subcores
    rpw = n // nw
    ch = min(rpw, 64)
    nch = rpw // ch
    nbuf = min(7, nch)
    mesh = plsc.VectorSubcoreMesh(core_axis_name="c", subcore_axis_name="s")
    idx3 = idx.reshape(nw, nch, ch)

    @functools.partial(
        pl.kernel, mesh=mesh,
        out_type=jax.ShapeDtypeStruct((n, d), jnp.float32),
        scratch_types=(
            [pltpu.VMEM((nch, ch), jnp.int32)]
            + [pltpu.VMEM((ch, d), jnp.float32) for _ in range(nbuf)]
            + [pltpu.SemaphoreType.DMA for _ in range(2 * nbuf)]
        ),
    )
    def k(tab_hbm, idx_hbm, out_hbm, idx_v, *rest):
        bufs = rest[:nbuf]
        gsems = rest[nbuf:2 * nbuf]
        ssems = rest[2 * nbuf:3 * nbuf]
        wid = lax.axis_index("s") * info.num_cores + lax.axis_index("c")
        base = wid * rpw
        pltpu.sync_copy(idx_hbm.at[wid], idx_v)
        gcp = {}
        scp = {}
        for ci in range(min(nbuf, nch)):
            gcp[ci] = pltpu.async_copy(tab_hbm.at[idx_v.at[ci]],
                                       bufs[ci % nbuf], gsems[ci % nbuf])
        for ci in range(nch):
            gcp[ci].wait()
            scp[ci] = pltpu.async_copy(
                bufs[ci % nbuf],
                out_hbm.at[pl.ds(base + ci * ch, ch)],
                ssems[ci % nbuf])
            nxt = ci + nbuf
            if nxt < nch:
                scp[ci].wait()
                gcp[nxt] = pltpu.async_copy(tab_hbm.at[idx_v.at[nxt]],
                                            bufs[nxt % nbuf], gsems[nxt % nbuf])
        for ci in range(max(0, nch - nbuf), nch):
            scp[ci].wait()

    return k(table, idx3)


def _k2_body(km, tm, vld, catp, f2, f3, pr, tmr, embcat, embev, aff,
             encW, encb, encg, encbe, wq, wk, wv,
             es_o, q_o, k_o, v_o):
    b = pl.program_id(0)
    kk = pl.program_id(1)

    @pl.when(vld[b, kk] == 1)
    def _():
        ccol = _col128(catp[0])
        lanef = lax.broadcasted_iota(jnp.int32, (BLK, 1024), 1).astype(jnp.float32)
        ohc = jnp.where(lanef == ccol, 1.0, 0.0)
        fcat = jnp.dot(ohc, embcat[...], preferred_element_type=jnp.float32,
                       precision=lax.Precision.HIGHEST)
        x = fcat + f2[0] + f3[0]
        x = x + embev[0]
        pc = _col128(pr[0])
        tc = _col128(tmr[0])
        x = x + pc * aff[0:1, :] + aff[1:2, :] + tc * aff[2:3, :] + aff[3:4, :]
        h = jnp.dot(x, encW[0], preferred_element_type=jnp.float32) + encb[0]
        h = jnp.maximum(_ln(h, encg[0], encbe[0]), 0.0)
        es_o[0] = h
        q_o[0] = jnp.dot(h, wq[0], preferred_element_type=jnp.float32)
        k_o[0] = jnp.dot(h, wk[0], preferred_element_type=jnp.float32)
        v_o[0] = jnp.dot(h, wv[0], preferred_element_type=jnp.float32)


def _k3_body(km, tm, bom, nkvm, slm, q_r, kf, vf, es_r, wo, g1, b1,
             x1_o, s_ref):
    b = pl.program_id(0)
    kk = pl.program_id(1)

    @pl.when(nkvm[b, kk] > 0)
    def _():
        bo = bom[b, kk]
        nkv = nkvm[b, kk]
        sl = slm[b, kk]
        q = q_r[0]
        kiota = lax.broadcasted_iota(jnp.int32, (BLK, BLK), 1)

        def score_body(j, _):
            kb = kf[0, pl.ds((bo + j) * BLK, BLK), :]
            kvvalid = (j * BLK + kiota) < sl
            for h in range(NH):
                qh = q[:, h * DH:(h + 1) * DH]
                kh = kb[:, h * DH:(h + 1) * DH]
                s = lax.dot_general(qh, kh, (((1,), (1,)), ((), ())),
                                    preferred_element_type=jnp.float32) * SCALE
                s_ref[h, j] = jnp.where(kvvalid, s, NEG)
            return 0

        lax.fori_loop(0, nkv, score_body, 0)

        outs = []
        for h in range(NH):
            def maxb(j, m):
                return jnp.maximum(m, jnp.max(s_ref[h, j], axis=1, keepdims=True))
            m = lax.fori_loop(0, nkv, maxb, jnp.full((BLK, 1), NEG, jnp.float32))

            def pdv(j, carry):
                den, o = carry
                pj = jnp.exp(s_ref[h, j] - m)
                den = den + jnp.sum(pj, axis=1, keepdims=True)
                vb = vf[0, pl.ds((bo + j) * BLK, BLK), h * DH:(h + 1) * DH]
                o = o + jnp.dot(pj, vb, preferred_element_type=jnp.float32)
                return den, o

            den, o = lax.fori_loop(
                0, nkv, pdv,
                (jnp.zeros((BLK, 1), jnp.float32),
                 jnp.zeros((BLK, DH), jnp.float32)))
            outs.append(o / den)

        attn = jnp.concatenate(outs, axis=1)
        o = jnp.dot(attn, wo[0], preferred_element_type=jnp.float32)
        x = es_r[0] + o
        x1_o[0] = _ln(x, g1[0], b1[0])


def _k4_body(km, tm, vld, vlm, fstm, x1_r, w1, bb1, w2, bb2, g2, be2,
             x2_o, pool_o):
    b = pl.program_id(0)
    kk = pl.program_id(1)

    @pl.when(vld[b, kk] == 1)
    def _():
        x1 = x1_r[0]
        h = jnp.dot(x1, w1[0], preferred_element_type=jnp.float32)
        h = jnp.maximum(h + bb1[0], 0.0)
        y = jnp.dot(h, w2[0], preferred_element_type=jnp.float32)
        y = y + bb2[0]
        x2 = _ln(x1 + y, g2[0], be2[0])
        x2_o[0] = x2
        riota = lax.broadcasted_iota(jnp.int32, (BLK, 1), 0)
        msk = (riota < vlm[b, kk]).astype(jnp.float32)
        ps = jnp.sum(x2 * msk, axis=0, keepdims=True)

        @pl.when(fstm[b, kk] == 1)
        def _():
            pool_o[0] = ps

        @pl.when(fstm[b, kk] == 0)
        def _():
            pool_o[0] = pool_o[0] + ps


def _logsig(x):
    return jnp.minimum(x, 0.0) - jnp.log(1.0 + jnp.exp(-jnp.abs(x)))


def _k6_body(pool, cntr, fw1, fb1, fg1, fbe1, fw2, fb2, fg2, fbe2,
             wcr, bcb, wcat, bcat, wprod, bprod, churn_b, catp, prodp,
             user_o, chl_o, cat_o, prod_o, scal_o):
    u = jnp.where(cntr[...] > 0, pool[...] / jnp.maximum(cntr[...], 1.0), 0.0)
    h = jnp.dot(u, fw1[...], preferred_element_type=jnp.float32) + fb1[...]
    h = jnp.maximum(_ln(h, fg1[...], fbe1[...]), 0.0)
    us = jnp.dot(h, fw2[...], preferred_element_type=jnp.float32) + fb2[...]
    us = jnp.tanh(_ln(us, fg2[...], fbe2[...]))
    user_o[...] = us

    chl = jnp.sum(us * wcr[...], axis=1, keepdims=True) + bcb[0:1, 0:1]
    lanes128 = lax.broadcasted_iota(jnp.int32, (B, BLK), 1)
    chl_o[...] = jnp.where(lanes128 == 0, chl, 0.0)

    cat = jnp.dot(us, wcat[...], preferred_element_type=jnp.float32) + bcat[...]
    cat_o[...] = cat
    prod = jnp.dot(us, wprod[...], preferred_element_type=jnp.float32) + bprod[...]
    prod_o[...] = prod

    churn_col = churn_b[...][:, 0:1]
    pw = jnp.where(jnp.sum(churn_col) > 0.0, 5.0, 1.0)
    tch = -(pw * churn_col * _logsig(chl) + (1.0 - churn_col) * _logsig(-chl))
    cl = jnp.sum(tch) / B

    ycat = (catp[...] > 0.0).astype(jnp.float32)
    mcat = (lanes128 < NCAT).astype(jnp.float32)
    tcat = -(ycat * _logsig(cat) + (1.0 - ycat) * _logsig(-cat)) * mcat
    catl = jnp.sum(tcat) / (B * NCAT)

    lanes1024 = lax.broadcasted_iota(jnp.int32, (B, 1024), 1)
    yprod = (prodp[...] > 0.0).astype(jnp.float32)
    mprod = (lanes1024 < NPROD).astype(jnp.float32)
    tprod = -(yprod * _logsig(prod) + (1.0 - yprod) * _logsig(-prod)) * mprod
    prodl = jnp.sum(tprod) / (B * NPROD)

    total = cl + 0.4 * catl + 0.4 * prodl
    total = jnp.where(jnp.isnan(total) | jnp.isinf(total), 100.0, total)
    slanes = lax.broadcasted_iota(jnp.int32, (1, BLK), 1)
    sc = jnp.where(slanes == 0, cl, 0.0)
    sc = jnp.where(slanes == 1, catl, sc)
    sc = jnp.where(slanes == 2, prodl, sc)
    sc = jnp.where(slanes == 3, total * 0.1, sc)
    scal_o[...] = sc


def _encode_half(et, categories, names, queries, prices, timestamps, p):
    B = et.shape[0]
    onehotf = (et[:, :, None] == jnp.arange(5)[None, None, :]).astype(jnp.float32)
    c = onehotf.sum(1).astype(jnp.int32)
    csum = jnp.cumsum(onehotf, axis=1)
    rank = (onehotf * csum).sum(-1) - 1.0
    nb = (c + BLK - 1) // BLK
    blk_end = jnp.cumsum(nb, axis=1)
    blk_off = blk_end - nb
    nblk = blk_end[:, -1]
    bo_tok = (onehotf * blk_off[:, None, :].astype(jnp.float32)).sum(-1)
    pos_map_f = bo_tok * BLK + rank
    pos_map = pos_map_f.astype(jnp.int32)

    kk = jnp.arange(NBLK)[None, :]
    k_eff = jnp.minimum(kk, (nblk - 1)[:, None])
    t_of = (k_eff[:, :, None] >= blk_end[:, None, :]).sum(-1).astype(jnp.int32)
    toh = (t_of[:, :, None] == jnp.arange(5)[None, None, :]).astype(jnp.int32)
    bo = (toh * blk_off[:, None, :]).sum(-1)
    nkv = (toh * nb[:, None, :]).sum(-1)
    seg_len = (toh * c[:, None, :]).sum(-1)
    blk_in_seg = k_eff - bo
    vlen = jnp.clip(seg_len - blk_in_seg * BLK, 0, BLK)
    validb = (kk < nblk[:, None]).astype(jnp.int32)
    firstb = ((blk_in_seg == 0) & (validb == 1)).astype(jnp.int32)
    nkv_g = (nkv * validb).astype(jnp.int32)

    km = k_eff.astype(jnp.int32)
    tmb = t_of
    bob = bo.astype(jnp.int32)
    slb = seg_len.astype(jnp.int32)
    vlb = vlen.astype(jnp.int32)

    dataT = jnp.stack([categories.astype(jnp.float32),
                       names.astype(jnp.float32),
                       queries.astype(jnp.float32),
                       prices, timestamps], axis=1)
    perm = pl.pallas_call(
        _k0_body,
        grid=(B,),
        in_specs=[pl.BlockSpec((1, 1, S), lambda b: (b, 0, 0)),
                  pl.BlockSpec((1, 5, S), lambda b: (b, 0, 0))],
        out_specs=pl.BlockSpec((1, 5, P), lambda b: (b, 0, 0)),
        out_shape=jax.ShapeDtypeStruct((B, 5, P), jnp.float32),
    )(pos_map.reshape(B, 1, S), dataT)
    cat_p = jnp.round(perm[:, 0]).astype(jnp.int32)
    name_p = jnp.round(perm[:, 1]).astype(jnp.int32)
    query_p = jnp.round(perm[:, 2]).astype(jnp.int32)
    price_p = perm[:, 3]
    time_p = perm[:, 4]

    f_name = _sc_gather_rows(p['emb_name'],
                             name_p.reshape(-1)).reshape(B, P, H)
    f_query = _sc_gather_rows(p['emb_query'],
                              query_p.reshape(-1)).reshape(B, P, H)
    embcat_pad = jnp.pad(p['emb_cat'], ((0, 1024 - 1000), (0, 0)))
    catp3 = perm[:, 0].reshape(B * NBLK, 1, BLK)

    aff = jnp.stack([p['w_price'], p['b_price'], p['w_time'], p['b_time']], 0)
    pr3 = price_p.reshape(B * NBLK, 1, BLK)
    tm3 = time_p.reshape(B * NBLK, 1, BLK)

    tok_spec = pl.BlockSpec((1, BLK, H), lambda b, k, km, tm, vd: (b, km[b, k], 0))
    sc_spec = pl.BlockSpec((1, 1, BLK),
                           lambda b, k, km, tm, vd: (b * NBLK + km[b, k], 0, 0))
    whh = pl.BlockSpec((1, H, H), lambda b, k, km, tm, vd: (tm[b, k], 0, 0))
    wh = pl.BlockSpec((1, 1, H), lambda b, k, km, tm, vd: (tm[b, k], 0, 0))

    es, q, k_, v = pl.pallas_call(
        _k2_body,
        grid_spec=pltpu.PrefetchScalarGridSpec(
            num_scalar_prefetch=3,
            grid=(B, NBLK),
            in_specs=[
                sc_spec, tok_spec, tok_spec, sc_spec, sc_spec,
                pl.BlockSpec((1024, H), lambda b, k, *r: (0, 0)),
                wh,
                pl.BlockSpec((4, H), lambda b, k, *r: (0, 0)),
                whh, wh, wh, wh,
                whh, whh, whh,
            ],
            out_specs=[tok_spec, tok_spec, tok_spec, tok_spec],
        ),
        out_shape=[jax.ShapeDtypeStruct((B, P, H), jnp.float32)] * 4,
    )(km, tmb, validb,
      catp3, f_name, f_query, pr3, tm3,
      embcat_pad,
      p['emb_event'].reshape(5, 1, H), aff,
      p['enc_W'], p['enc_b'].reshape(5, 1, H), p['enc_g'].reshape(5, 1, H),
      p['enc_be'].reshape(5, 1, H),
      p['Wq'], p['Wk'], p['Wv'])

    tok5 = pl.BlockSpec((1, BLK, H),
                        lambda b, k, km, tm, bo, nk, sl: (b, km[b, k], 0))
    row5 = pl.BlockSpec((1, P, H),
                        lambda b, k, km, tm, bo, nk, sl: (b, 0, 0))

    x1 = pl.pallas_call(
        _k3_body,
        grid_spec=pltpu.PrefetchScalarGridSpec(
            num_scalar_prefetch=5,
            grid=(B, NBLK),
            in_specs=[tok5, row5, row5, tok5,
                      pl.BlockSpec((1, H, H),
                                   lambda b, k, km, tm, bo, nk, sl: (tm[b, k], 0, 0)),
                      pl.BlockSpec((1, 1, H),
                                   lambda b, k, km, tm, bo, nk, sl: (tm[b, k], 0, 0)),
                      pl.BlockSpec((1, 1, H),
                                   lambda b, k, km, tm, bo, nk, sl: (tm[b, k], 0, 0))],
            out_specs=[tok5],
            scratch_shapes=[pltpu.VMEM((NH, 4, BLK, BLK), jnp.float32)],
        ),
        out_shape=[jax.ShapeDtypeStruct((B, P, H), jnp.float32)],
    )(km, tmb, bob, nkv_g, slb,
      q, k_, v, es, p['Wo'], p['ln1_g'].reshape(5, 1, H),
      p['ln1_b'].reshape(5, 1, H))[0]

    tok4 = pl.BlockSpec((1, BLK, H),
                        lambda b, k, km, tm, vd, vl, fs: (b, km[b, k], 0))
    x2, pool = pl.pallas_call(
        _k4_body,
        grid_spec=pltpu.PrefetchScalarGridSpec(
            num_scalar_prefetch=5,
            grid=(B, NBLK),
            in_specs=[
                tok4,
                pl.BlockSpec((1, H, 4 * H),
                             lambda b, k, km, tm, vd, vl, fs: (tm[b, k], 0, 0)),
                pl.BlockSpec((1, 1, 4 * H),
                             lambda b, k, km, tm, vd, vl, fs: (tm[b, k], 0, 0)),
                pl.BlockSpec((1, 4 * H, H),
                             lambda b, k, km, tm, vd, vl, fs: (tm[b, k], 0, 0)),
                pl.BlockSpec((1, 1, H),
                             lambda b, k, km, tm, vd, vl, fs: (tm[b, k], 0, 0)),
                pl.BlockSpec((1, 1, H),
                             lambda b, k, km, tm, vd, vl, fs: (tm[b, k], 0, 0)),
                pl.BlockSpec((1, 1, H),
                             lambda b, k, km, tm, vd, vl, fs: (tm[b, k], 0, 0)),
            ],
            out_specs=[
                tok4,
                pl.BlockSpec((1, 1, H),
                             lambda b, k, km, tm, vd, vl, fs: (b * 5 + tm[b, k], 0, 0)),
            ],
        ),
        out_shape=[jax.ShapeDtypeStruct((B, P, H), jnp.float32),
                   jax.ShapeDtypeStruct((B * 5, 1, H), jnp.float32)],
    )(km, tmb, validb, vlb, firstb,
      x1, p['f_W1'], p['f_b1'].reshape(5, 1, 4 * H), p['f_W2'],
      p['f_b2'].reshape(5, 1, H), p['ln2_g'].reshape(5, 1, H),
      p['ln2_b'].reshape(5, 1, H))

    tidx = (jnp.arange(B)[:, None] * P + pos_map).reshape(-1).astype(jnp.int32)
    temporal = _sc_gather_rows(x2.reshape(B * P, H), tidx).reshape(B, S, H)

    return pool, c, temporal


def kernel(event_types, categories, prices, names, queries, timestamps,
           mask, churn, category_propensity, product_propensity,
           client_id, params):
    p = params
    et = event_types.astype(jnp.int32)
    HB = 4
    parts = [_encode_half(et[i:i + HB], categories[i:i + HB],
                          names[i:i + HB], queries[i:i + HB],
                          prices[i:i + HB], timestamps[i:i + HB], p)
             for i in range(0, 16, HB)]
    B = 16
    pool = jnp.concatenate([x[0] for x in parts], axis=0)
    c = jnp.concatenate([x[1] for x in parts], axis=0)
    temporal = jnp.concatenate([x[2] for x in parts], axis=0)

    pool2 = pool.reshape(B, 5 * H)
    cntr = jnp.repeat(c.astype(jnp.float32), H, axis=1)
    churn_b = jnp.broadcast_to(churn[:, None], (B, BLK))
    catp = jnp.pad(category_propensity, ((0, 0), (0, BLK - NCAT)),
                   constant_values=-1.0)
    prodp = jnp.pad(product_propensity, ((0, 0), (0, 1024 - NPROD)),
                    constant_values=-1.0)
    wcat = jnp.pad(p['Wcat'], ((0, 0), (0, BLK - NCAT)))
    bcat = jnp.pad(p['bcat'], (0, BLK - NCAT)).reshape(1, BLK)
    wprod = jnp.pad(p['Wprod'], ((0, 0), (0, 1024 - NPROD)))
    bprod = jnp.pad(p['bprod'], (0, 1024 - NPROD)).reshape(1, 1024)
    wcr = p['Wc'].reshape(1, H)
    bcb = jnp.broadcast_to(p['bc'].reshape(1, 1), (1, BLK))

    user, chl_o, cat_o, prod_o, scal = pl.pallas_call(
        _k6_body,
        out_shape=[
            jax.ShapeDtypeStruct((B, H), jnp.float32),
            jax.ShapeDtypeStruct((B, BLK), jnp.float32),
            jax.ShapeDtypeStruct((B, BLK), jnp.float32),
            jax.ShapeDtypeStruct((B, 1024), jnp.float32),
            jax.ShapeDtypeStruct((1, BLK), jnp.float32),
        ],
    )(pool2, cntr,
      p['fus_W1'], p['fus_b1'].reshape(1, 2 * H), p['fus_g1'].reshape(1, 2 * H),
      p['fus_be1'].reshape(1, 2 * H),
      p['fus_W2'], p['fus_b2'].reshape(1, H), p['fus_g2'].reshape(1, H),
      p['fus_be2'].reshape(1, H),
      wcr, bcb, wcat, bcat, wprod, bprod, churn_b, catp, prodp)

    churn_logits = chl_o[:, 0]
    cat_logits = cat_o[:, :NCAT]
    prod_logits = prod_o[:, :NPROD]
    return (user, temporal, churn_logits, cat_logits, prod_logits,
            scal[0, 0], scal[0, 1], scal[0, 2], scal[0, 3])
